# baseline (speedup 1.0000x reference)
import functools

import jax
import jax.numpy as jnp
from jax import lax
from jax.experimental import pallas as pl
from jax.experimental.pallas import tpu as pltpu
from jax.experimental.pallas import tpu_sc as plsc

VOCAB = 1000
B, T = 1024, 50
TPAD = 56
NC, NS = 2, 16
NW = NC * NS
ROWS_PW = B // NW

_mesh = plsc.VectorSubcoreMesh(core_axis_name="c", subcore_axis_name="s")


@functools.partial(
    pl.kernel,
    mesh=_mesh,
    out_type=jax.ShapeDtypeStruct((B, T, VOCAB), jnp.float32),
    scratch_types=[
        pltpu.VMEM((ROWS_PW, TPAD), jnp.int32),
        [pltpu.VMEM((TPAD, VOCAB), jnp.float32) for _ in range(2)],
        [pltpu.SemaphoreType.DMA for _ in range(2)],
    ],
    compiler_params=pltpu.CompilerParams(use_tc_tiling_on_sc=False),
)
def _gather_kernel(table_hbm, idx_hbm, out_hbm, idx_v, bufs, sems):
    sid = lax.axis_index("s")
    wid = sid * NC + lax.axis_index("c")
    base = wid * ROWS_PW
    pltpu.sync_copy(idx_hbm.at[pl.ds(base, ROWS_PW)], idx_v)

    def start_gather(ch, b):
        pltpu.async_copy(
            table_hbm.at[idx_v.at[ch]], bufs[b], sems[b]
        )

    start_gather(0, 0)

    def step(i, carry):
        for b in range(2):
            ch = i * 2 + b
            pltpu.make_async_copy(
                table_hbm.at[pl.ds(0, TPAD)], bufs[b], sems[b]
            ).wait()

            @pl.when(ch + 1 < ROWS_PW)
            def _():
                start_gather(ch + 1, 1 - b)

            pltpu.sync_copy(bufs[b].at[pl.ds(0, T)], out_hbm.at[base + ch])
        return carry

    lax.fori_loop(0, ROWS_PW // 2, step, 0)


def kernel(idx, table):
    idx_p = jnp.pad(idx.astype(jnp.int32), ((0, 0), (0, TPAD - T)))
    return _gather_kernel(table, idx_p)


# speedup vs baseline: 1.3159x; 1.3159x over previous
import functools

import jax
import jax.numpy as jnp
from jax import lax
from jax.experimental import pallas as pl
from jax.experimental.pallas import tpu as pltpu
from jax.experimental.pallas import tpu_sc as plsc

VOCAB = 1000
B, T = 1024, 50
TPAD = 56
CMAIN = 896
CTAIL = VOCAB - CMAIN
TMAIN = 48
NC, NS = 2, 16
NW = NC * NS
ROWS_PW = B // NW

_mesh = plsc.VectorSubcoreMesh(core_axis_name="c", subcore_axis_name="s")


@functools.partial(
    pl.kernel,
    mesh=_mesh,
    out_type=jax.ShapeDtypeStruct((B, T, VOCAB), jnp.float32),
    scratch_types=[
        pltpu.VMEM((ROWS_PW * TPAD,), jnp.int32),
        [pltpu.VMEM((T, VOCAB), jnp.float32) for _ in range(2)],
        pltpu.VMEM((8, 1024), jnp.float32),
        pltpu.VMEM((TMAIN, 128), jnp.float32),
        [pltpu.SemaphoreType.DMA for _ in range(2)],
        pltpu.SemaphoreType.DMA,
        pltpu.SemaphoreType.DMA,
        [pltpu.SemaphoreType.DMA for _ in range(2)],
    ],
    compiler_params=pltpu.CompilerParams(needs_layout_passes=False),
)
def _gather_kernel(
    table_a, table_b, table_p, idx_hbm, out_hbm,
    idx_v, bufw, bufg2, buft, ga, gb, gt, wsem,
):
    sid = lax.axis_index("s")
    wid = sid * NC + lax.axis_index("c")
    base = wid * ROWS_PW
    pltpu.sync_copy(idx_hbm.at[pl.ds(base * TPAD, ROWS_PW * TPAD)], idx_v)

    def start_gathers(ch, p):
        off = ch * TPAD
        pltpu.async_copy(
            table_a.at[idx_v.at[pl.ds(off, TMAIN)]],
            bufw[p].at[pl.ds(0, TMAIN), pl.ds(0, CMAIN)],
            ga[p],
        )
        pltpu.async_copy(
            table_p.at[idx_v.at[pl.ds(off + TMAIN, 8)]], bufg2, gb
        )
        pltpu.async_copy(
            table_b.at[idx_v.at[pl.ds(off, TMAIN)]], buft, gt
        )

    def edge_copies(p):
        bw = bufw[p]
        stray_cols = 992 + lax.broadcasted_iota(jnp.int32, (16,), 0)
        stray_mask = stray_cols < VOCAB

        def stray(row, vals):
            plsc.store_scatter(
                bw, [jnp.full((16,), row, jnp.int32), stray_cols], vals,
                mask=stray_mask,
            )

        def tail_row(r, c):
            for k in range(CTAIL // 16):
                bw[r, pl.ds(CMAIN + 16 * k, 16)] = buft[r, pl.ds(16 * k, 16)]
            stray(r, buft[r, pl.ds(96, 16)])
            return c

        lax.fori_loop(0, TMAIN, tail_row, 0)

        def full_row(r, c):
            for k in range(62):
                bw[TMAIN + r, pl.ds(16 * k, 16)] = bufg2[r, pl.ds(16 * k, 16)]
            stray(TMAIN + r, bufg2[r, pl.ds(992, 16)])
            return c

        lax.fori_loop(0, T - TMAIN, full_row, 0)

    def wait_gathers(p):
        pltpu.make_async_copy(
            table_a.at[pl.ds(0, TMAIN)],
            bufw[p].at[pl.ds(0, TMAIN), pl.ds(0, CMAIN)],
            ga[p],
        ).wait()
        pltpu.make_async_copy(table_p.at[pl.ds(0, 8)], bufg2, gb).wait()
        pltpu.make_async_copy(table_b.at[pl.ds(0, TMAIN)], buft, gt).wait()

    def wait_write(p):
        pltpu.make_async_copy(bufw[p], out_hbm.at[base], wsem[p]).wait()

    start_gathers(0, 0)

    def step(i, carry):
        for p in range(2):
            ch = i * 2 + p
            wait_gathers(p)
            edge_copies(p)

            @pl.when(ch >= 2)
            def _():
                wait_write(1 - p)

            @pl.when(ch + 1 < ROWS_PW)
            def _():
                start_gathers(ch + 1, 1 - p)

            pltpu.async_copy(bufw[p], out_hbm.at[base + ch], wsem[p])
        return carry

    lax.fori_loop(0, ROWS_PW // 2, step, 0)
    wait_write(0)
    wait_write(1)


def kernel(idx, table):
    idx_p = jnp.pad(idx.astype(jnp.int32), ((0, 0), (0, TPAD - T)))
    table_a = table[:, :CMAIN]
    table_b = jnp.pad(table[:, CMAIN:], ((0, 0), (0, 128 - CTAIL)))
    table_p = jnp.pad(table, ((0, 0), (0, 1024 - VOCAB)))
    return _gather_kernel(table_a, table_b, table_p, idx_p.reshape(-1))
